# Initial kernel scaffold; baseline (speedup 1.0000x reference)
import jax
import jax.numpy as jnp
from jax.experimental import pallas as pl


def kernel(x_user, x_item, edge_index_follows, edge_index_clicks, W_follows, W_clicked):
    raise NotImplementedError("write your pallas kernel here")


# speedup vs baseline: 4.5045x; 4.5045x over previous
import math

import jax
import jax.numpy as jnp
from jax import lax
from jax.experimental import pallas as pl
from jax.experimental.pallas import tpu as pltpu
from jax.experimental.pallas import tpu_sc as plsc

_NS = 16
_NC = 2
_B = 128


def _build_sc_kernel(n_pad, stripe, nb, d, interpret=False):
  mesh = plsc.VectorSubcoreMesh(
      core_axis_name="c", subcore_axis_name="s",
      num_cores=_NC, num_subcores=_NS)
  chunks = stripe // _B

  def body(x_hbm, src_hbm, dst_hbm, sums_hbm, degs_hbm,
           src_v, dst_v, rows_v, zbuf, dvec, ones_v, acc_sh, deg_sh, sem):
    cid = lax.axis_index("c")
    sid = lax.axis_index("s")
    w = cid * _NS + sid
    base = sid * stripe

    def _zrow(i, c):
      for k in range(d // 16):
        zbuf[i, pl.ds(k * 16, 16)] = jnp.zeros((16,), jnp.float32)
      return c
    lax.fori_loop(0, _B, _zrow, 0)

    def _zdeg(i, c):
      dvec[pl.ds(i * 16, 16)] = jnp.zeros((16,), jnp.float32)
      return c
    lax.fori_loop(0, stripe // 16, _zdeg, 0)

    for k in range(_B // 16):
      ones_v[pl.ds(k * 16, 16)] = jnp.ones((16,), jnp.float32)

    for k in range(chunks):
      pltpu.sync_copy(zbuf, acc_sh.at[pl.ds(base + k * _B, _B)])
    pltpu.sync_copy(dvec, deg_sh.at[pl.ds(base, stripe)])
    plsc.subcore_barrier()

    def _step(j, c):
      pltpu.sync_copy(src_hbm.at[w, j], src_v)
      pltpu.sync_copy(dst_hbm.at[w, j], dst_v)
      pltpu.async_copy(x_hbm.at[src_v], rows_v, sem).wait()
      pltpu.sync_copy(rows_v, acc_sh.at[dst_v], add=True)
      pltpu.sync_copy(ones_v, deg_sh.at[dst_v], add=True)
      return c
    lax.fori_loop(0, nb, _step, 0)
    plsc.subcore_barrier()

    out_base = cid * n_pad + base
    for k in range(chunks):
      pltpu.sync_copy(acc_sh.at[pl.ds(base + k * _B, _B)], zbuf)
      pltpu.sync_copy(zbuf, sums_hbm.at[pl.ds(out_base + k * _B, _B)])
    pltpu.sync_copy(deg_sh.at[pl.ds(base, stripe)], dvec)
    pltpu.sync_copy(dvec, degs_hbm.at[pl.ds(out_base, stripe)])

  return pl.kernel(
      body,
      out_type=(
          jax.ShapeDtypeStruct((_NC * n_pad, d), jnp.float32),
          jax.ShapeDtypeStruct((_NC * n_pad,), jnp.float32),
      ),
      mesh=mesh,
      scratch_types=[
          pltpu.VMEM((_B,), jnp.int32),
          pltpu.VMEM((_B,), jnp.int32),
          pltpu.VMEM((_B, d), jnp.float32),
          pltpu.VMEM((_B, d), jnp.float32),
          pltpu.VMEM((stripe,), jnp.float32),
          pltpu.VMEM((_B,), jnp.float32),
          pltpu.VMEM_SHARED((n_pad, d), jnp.float32),
          pltpu.VMEM_SHARED((n_pad,), jnp.float32),
          pltpu.SemaphoreType.DMA,
      ],
      interpret=interpret,
  )


def _build_tc_kernel(n_dst, d, rows, interpret=False):
  grid = (n_dst // rows,)

  def body(sf_ref, sc_ref, df_ref, dc_ref, wf_ref, wc_ref, o_ref):
    sf = sf_ref[0]
    sc_ = sc_ref[0]
    df = df_ref[0]
    dc = dc_ref[0]
    hf = sf * (1.0 / jnp.maximum(df, 1.0))
    hc = sc_ * (1.0 / jnp.maximum(dc, 1.0))
    o_ref[...] = (
        jnp.dot(hf, wf_ref[...], preferred_element_type=jnp.float32)
        + jnp.dot(hc, wc_ref[...], preferred_element_type=jnp.float32))

  return pl.pallas_call(
      body,
      grid=grid,
      in_specs=[
          pl.BlockSpec((1, rows, d), lambda i: (0, i, 0)),
          pl.BlockSpec((1, rows, d), lambda i: (1, i, 0)),
          pl.BlockSpec((1, rows, 1), lambda i: (0, i, 0)),
          pl.BlockSpec((1, rows, 1), lambda i: (1, i, 0)),
          pl.BlockSpec((d, d), lambda i: (0, 0)),
          pl.BlockSpec((d, d), lambda i: (0, 0)),
      ],
      out_specs=pl.BlockSpec((rows, d), lambda i: (i, 0)),
      out_shape=jax.ShapeDtypeStruct((n_dst, d), jnp.float32),
      interpret=interpret,
  )


def _row_block(n):
  for r in range(min(512, n), 0, -8):
    if n % r == 0:
      return r
  return 8


def kernel(x_user, x_item, edge_index_follows, edge_index_clicks,
           W_follows, W_clicked):
  n_user, d = x_user.shape
  e = edge_index_follows.shape[1]
  stripe = math.ceil((n_user + 1) / (_NS * _B)) * _B
  n_pad = _NS * stripe
  nb = math.ceil(e / (_NS * _B))
  e_pad = _NS * _B * nb

  x_cat = jnp.concatenate([x_user, x_item], axis=0)

  def prep(src, dst):
    ps = jnp.concatenate([src, jnp.zeros((e_pad - e,), jnp.int32)])
    pd = jnp.concatenate(
        [dst, jnp.full((e_pad - e,), n_user, jnp.int32)])
    return ps, pd

  sf, dstf = prep(edge_index_follows[0], edge_index_follows[1])
  sc_, dstc = prep(edge_index_clicks[0] + n_user, edge_index_clicks[1])
  src_all = jnp.stack([sf, sc_]).reshape(_NC * _NS, nb, _B)
  dst_all = jnp.stack([dstf, dstc]).reshape(_NC * _NS, nb, _B)

  sums, degs = _build_sc_kernel(n_pad, stripe, nb, d)(
      x_cat, src_all, dst_all)
  sums3 = sums.reshape(_NC, n_pad, d)
  degs3 = degs.reshape(_NC, n_pad, 1)

  rows = _row_block(n_user)
  return _build_tc_kernel(n_user, d, rows)(
      sums3, sums3, degs3, degs3, W_follows, W_clicked)
